# baseline (speedup 1.0000x reference)
import jax
import jax.numpy as jnp
from jax.experimental import pallas as pl

H = 4
C = 64
HID = 64
B = 8


def _leaky(x, s):
    return jnp.where(x >= 0, x, s * x)


def _ln(x, g, b, eps=1e-5):
    m = x.mean(axis=-1, keepdims=True)
    v = ((x - m) ** 2).mean(axis=-1, keepdims=True)
    return (x - m) / jnp.sqrt(v + eps) * g + b


def _gatv2(x_src, x_dst, ei, Wl, Wr, att, bias, Wres):
    n_dst = x_dst.shape[0]
    src, dst = ei[0], ei[1]
    xl = (x_src @ Wl).reshape(x_src.shape[0], H, C)
    xr = (x_dst @ Wr).reshape(n_dst, H, C)
    xi = xl[src]
    xj = xr[dst]
    e = _leaky(xi + xj, 0.2)
    logits = (e * att[None, :, :]).sum(-1)
    mx = jax.ops.segment_max(logits, dst, num_segments=n_dst)
    mx = jnp.where(jnp.isfinite(mx), mx, 0.0)
    ex = jnp.exp(logits - mx[dst])
    den = jax.ops.segment_sum(ex, dst, num_segments=n_dst)
    alpha = ex / (den[dst] + 1e-16)
    out = jax.ops.segment_sum(xi * alpha[..., None], dst, num_segments=n_dst)
    out = out.mean(axis=1)
    out = out + x_dst @ Wres
    out = out + bias
    return out


def _copy_k(x_ref, o_ref):
    o_ref[...] = x_ref[...]


def kernel(x_tasks, x_data, params, read_edge_index, read_edge_attr, tt_edge_index, b_tasks):
    p = params
    xt = _leaky(_ln(x_tasks @ p['stem_t_W'] + p['stem_t_b'], p['stem_t_g'], p['stem_t_beta']), 0.01)
    xd = _leaky(_ln(x_data @ p['stem_d_W'] + p['stem_d_b'], p['stem_d_g'], p['stem_d_beta']), 0.01)
    mask = read_edge_attr[:, 0] != 0
    ei = jnp.stack([read_edge_index[0],
                    jnp.where(mask, read_edge_index[1], x_tasks.shape[0])])
    data_fused_tasks = _gatv2(xd, xt, ei, p['td_Wl'], p['td_Wr'], p['td_att'], p['td_bias'], p['td_Wres'])
    tasks_fused_tasks = _gatv2(xt, xt, tt_edge_index, p['tt_Wl'], p['tt_Wr'], p['tt_att'], p['tt_bias'], p['tt_Wres'])
    x_data_updated = _leaky(_ln(data_fused_tasks, p['ln1_g'], p['ln1_b']), 0.01)
    x_tasks_updated = _leaky(_ln(tasks_fused_tasks, p['ln2_g'], p['ln2_b']), 0.01)
    x_fused = jnp.concatenate([xt, x_tasks_updated, x_data_updated], axis=-1)
    cnt = jax.ops.segment_sum(jnp.ones((x_fused.shape[0],), dtype=x_fused.dtype), b_tasks, num_segments=B)
    s = jax.ops.segment_sum(x_fused, b_tasks, num_segments=B)
    global_fused = s / jnp.clip(cnt, 1.0)[:, None]
    return pl.pallas_call(
        _copy_k, out_shape=jax.ShapeDtypeStruct(global_fused.shape, global_fused.dtype)
    )(global_fused)


# speedup vs baseline: 11.3710x; 11.3710x over previous
import functools

import jax
import jax.numpy as jnp
import numpy as np
from jax import lax
from jax.experimental import pallas as pl
from jax.experimental.pallas import tpu as pltpu
from jax.experimental.pallas import tpu_sc as plsc

H = 4
C = 64
HID = 64
B = 8
N = 50000
F = 64
E = 800000

NW = 32
G = 64
EPW = 25024
E_PAD = NW * EPW
NCH = EPW // G
NP = 51200
RPS = NP // 16
DEN_F = 4 * NP
ZS = DEN_F // 16

_f32 = jnp.float32
_i32 = jnp.int32


def _leaky(x, s):
    return jnp.maximum(x, s * x)


def _ln(x, g, b, eps=1e-5):
    m = x.mean(axis=-1, keepdims=True)
    v = ((x - m) ** 2).mean(axis=-1, keepdims=True)
    return (x - m) / jnp.sqrt(v + eps) * g + b


_q = np.arange(H * C)
_PERM = ((_q % 128) // 32) * 64 + (_q // 128) * 32 + (_q % 32)


def _tc1_body(xt_in, xd_in, stW, stb, stg, stbe, sdW, sdb, sdg, sdbe,
              wl_td, wr_td, wl_tt, wr_tt,
              xt_o, tdlo_o, tdhi_o, tdxr_o, ttlo_o, tthi_o, ttxr_o):
    xt = _leaky(_ln(jnp.dot(xt_in[...], stW[...]) + stb[...], stg[...], stbe[...]), 0.01)
    xd = _leaky(_ln(jnp.dot(xd_in[...], sdW[...]) + sdb[...], sdg[...], sdbe[...]), 0.01)
    xl_td = jnp.dot(xd, wl_td[...])
    xl_tt = jnp.dot(xt, wl_tt[...])
    xt_o[...] = xt
    tdlo_o[...] = xl_td[:, :128]
    tdhi_o[...] = xl_td[:, 128:]
    tdxr_o[...] = jnp.dot(xt, wr_td[...])
    ttlo_o[...] = xl_tt[:, :128]
    tthi_o[...] = xl_tt[:, 128:]
    ttxr_o[...] = jnp.dot(xt, wr_tt[...])


def _tc1(x_tasks, x_data, p):
    blk = 1000
    grid = N // blk
    row = lambda shape: pl.BlockSpec(shape, lambda i: (i, 0))
    full = lambda shape: pl.BlockSpec(shape, lambda i: (0, 0))
    outs = pl.pallas_call(
        _tc1_body,
        grid=(grid,),
        in_specs=[row((blk, F)), row((blk, F)),
                  full((F, HID)), full((1, HID)), full((1, HID)), full((1, HID)),
                  full((F, HID)), full((1, HID)), full((1, HID)), full((1, HID)),
                  full((HID, 256)), full((HID, 256)), full((HID, 256)), full((HID, 256))],
        out_specs=[row((blk, 64)), row((blk, 128)), row((blk, 128)), row((blk, 256)),
                   row((blk, 128)), row((blk, 128)), row((blk, 256))],
        out_shape=[jax.ShapeDtypeStruct((N, 64), _f32),
                   jax.ShapeDtypeStruct((N, 128), _f32),
                   jax.ShapeDtypeStruct((N, 128), _f32),
                   jax.ShapeDtypeStruct((N, 256), _f32),
                   jax.ShapeDtypeStruct((N, 128), _f32),
                   jax.ShapeDtypeStruct((N, 128), _f32),
                   jax.ShapeDtypeStruct((N, 256), _f32)],
    )(x_tasks, x_data,
      p['stem_t_W'], p['stem_t_b'].reshape(1, -1), p['stem_t_g'].reshape(1, -1),
      p['stem_t_beta'].reshape(1, -1),
      p['stem_d_W'], p['stem_d_b'].reshape(1, -1), p['stem_d_g'].reshape(1, -1),
      p['stem_d_beta'].reshape(1, -1),
      p['td_Wl'][:, _PERM], p['td_Wr'][:, _PERM],
      p['tt_Wl'][:, _PERM], p['tt_Wr'][:, _PERM])
    return outs


def _sc_a_body(src_h, dst_h, xllo_h, xlhi_h, xr_h, att_h, zden_h,
               ex_o, den_o,
               src_v, dst_v, xi_lo, xi_hi, xj, lg, idx_all, att_v, stg, den_s):
    core = lax.axis_index("c")
    sub = lax.axis_index("s")
    wid = sub * 2 + core
    pltpu.sync_copy(att_h, att_v)
    pltpu.sync_copy(zden_h, den_s.at[pl.ds(sub * ZS, ZS)])
    plsc.subcore_barrier()
    base_w = wid * EPW

    def chunk(ci, carry):
        base = base_w + ci * G
        pltpu.sync_copy(src_h.at[pl.ds(base, G)], src_v)
        pltpu.sync_copy(dst_h.at[pl.ds(base, G)], dst_v)
        pltpu.sync_copy(xllo_h.at[src_v], xi_lo)
        pltpu.sync_copy(xlhi_h.at[src_v], xi_hi)
        pltpu.sync_copy(xr_h.at[dst_v], xj)

        def group(g, c2):
            for i in range(16):
                e = 16 * g + i
                for h in range(H):
                    acc = None
                    for jj in range(2):
                        j = 2 * h + jj
                        for half, buf in ((0, xi_lo), (1, xi_hi)):
                            a = buf[e, pl.ds(16 * j, 16)]
                            bv = xj[e, pl.ds(128 * half + 16 * j, 16)]
                            s = a + bv
                            lk = jnp.maximum(s, 0.2 * s)
                            t = att_v[half * 8 + j, :] * lk
                            acc = t if acc is None else acc + t
                    stg[pl.ds(h * 256 + 16 * i, 16)] = acc
            for h in range(H):
                tot = None
                for j in range(16):
                    idxv = lax.iota(_i32, 16) * 16 + (h * 256 + j)
                    t = plsc.load_gather(stg, [idxv])
                    tot = t if tot is None else tot + t
                lg[pl.ds(h * G + 16 * g, 16)] = tot
            return c2

        lax.fori_loop(0, G // 16, group, 0)
        for t in range(16):
            v = lg[pl.ds(16 * t, 16)]
            lg[pl.ds(16 * t, 16)] = jnp.exp(v)
        for t in range(16):
            dvec = dst_v[pl.ds((t % 4) * 16, 16)]
            idx_all[pl.ds(16 * t, 16)] = dvec + (t // 4) * NP
        pltpu.sync_copy(lg, ex_o.at[pl.ds(base * 4, 4 * G)])
        pltpu.sync_copy(lg, den_s.at[idx_all], add=True)
        return carry

    lax.fori_loop(0, NCH, chunk, 0)
    plsc.subcore_barrier()
    pltpu.sync_copy(den_s.at[pl.ds(sub * ZS, ZS)], den_o.at[core, pl.ds(sub * ZS, ZS)])


def _sc_a(src, dst, xl_lo, xl_hi, xr_pad, att_perm, zden):
    mesh = plsc.VectorSubcoreMesh(core_axis_name="c", subcore_axis_name="s")
    k = pl.kernel(
        _sc_a_body,
        out_type=(jax.ShapeDtypeStruct((4 * E_PAD,), _f32),
                  jax.ShapeDtypeStruct((2, DEN_F), _f32)),
        mesh=mesh,
        scratch_types=[
            pltpu.VMEM((G,), _i32),
            pltpu.VMEM((G,), _i32),
            pltpu.VMEM((G, 128), _f32),
            pltpu.VMEM((G, 128), _f32),
            pltpu.VMEM((G, 256), _f32),
            pltpu.VMEM((4 * G,), _f32),
            pltpu.VMEM((4 * G,), _i32),
            pltpu.VMEM((16, 16), _f32),
            pltpu.VMEM((1024,), _f32),
            pltpu.VMEM_SHARED((DEN_F,), _f32),
        ],
        compiler_params=pltpu.CompilerParams(needs_layout_passes=False),
    )
    return k(src, dst, xl_lo, xl_hi, xr_pad, att_perm, zden)


def _splat(v, i):
    idx = jnp.full((16, 1), i, _i32)
    dn = lax.GatherDimensionNumbers(offset_dims=(), collapsed_slice_dims=(0,),
                                    start_index_map=(0,))
    return lax.gather(v, idx, dn, (1,),
                      mode=lax.GatherScatterMode.PROMISE_IN_BOUNDS)


NPW = NP * 32
SL2 = NPW // 16


def _sc_b_body(src_h, dst_h, xl_h, ex_h, rden_h, zout_h,
               out_o,
               src_v, dst_v, xi, exv, rdi, rdv, av, val2, idx2, out_s):
    core = lax.axis_index("c")
    sub = lax.axis_index("s")
    wid = sub * 2 + core
    pltpu.sync_copy(zout_h, out_s.at[pl.ds(sub * SL2, SL2)])
    plsc.subcore_barrier()
    base_w = wid * EPW

    def chunk(ci, carry):
        base = base_w + ci * G
        pltpu.sync_copy(src_h.at[pl.ds(base, G)], src_v)
        pltpu.sync_copy(dst_h.at[pl.ds(base, G)], dst_v)
        pltpu.sync_copy(xl_h.at[src_v], xi)
        pltpu.sync_copy(ex_h.at[pl.ds(base * 4, 4 * G)], exv)
        for t in range(16):
            dvec = dst_v[pl.ds((t % 4) * 16, 16)]
            rdi[pl.ds(16 * t, 16)] = dvec + (t // 4) * NP
        pltpu.sync_copy(rden_h.at[rdi], rdv)
        for t in range(16):
            av[pl.ds(16 * t, 16)] = exv[pl.ds(16 * t, 16)] * rdv[pl.ds(16 * t, 16)]
        iot = lax.iota(_i32, 16)

        def group(g, c2):
            vh = [av[pl.ds(h * G + 16 * g, 16)] for h in range(H)]
            dvec = dst_v[pl.ds(16 * g, 16)]
            for i in range(16):
                e = 16 * g + i
                sp = [_splat(vh[h], i) for h in range(H)]
                d32 = _splat(dvec, i) * 32
                for jj in range(2):
                    acc = None
                    for h in range(H):
                        t = sp[h] * xi[e, pl.ds(h * 32 + 16 * jj, 16)]
                        acc = t if acc is None else acc + t
                    val2[pl.ds(e * 32 + 16 * jj, 16)] = acc
                    idx2[pl.ds(e * 32 + 16 * jj, 16)] = d32 + (iot + 16 * jj)
            return c2

        lax.fori_loop(0, G // 16, group, 0)
        pltpu.sync_copy(val2, out_s.at[idx2], add=True)
        return carry

    lax.fori_loop(0, NCH, chunk, 0)
    plsc.subcore_barrier()
    pltpu.sync_copy(out_s.at[pl.ds(sub * SL2, SL2)],
                    out_o.at[pl.ds(core * NPW + sub * SL2, SL2)])


def _sc_b(src, dst, xl_half, ex_blob, rden, zout):
    mesh = plsc.VectorSubcoreMesh(core_axis_name="c", subcore_axis_name="s")
    k = pl.kernel(
        _sc_b_body,
        out_type=jax.ShapeDtypeStruct((2 * NPW,), _f32),
        mesh=mesh,
        scratch_types=[
            pltpu.VMEM((G,), _i32),
            pltpu.VMEM((G,), _i32),
            pltpu.VMEM((G, 128), _f32),
            pltpu.VMEM((4 * G,), _f32),
            pltpu.VMEM((4 * G,), _i32),
            pltpu.VMEM((4 * G,), _f32),
            pltpu.VMEM((4 * G,), _f32),
            pltpu.VMEM((32 * G,), _f32),
            pltpu.VMEM((32 * G,), _i32),
            pltpu.VMEM_SHARED((NPW,), _f32),
        ],
        compiler_params=pltpu.CompilerParams(needs_layout_passes=False),
    )
    return k(src, dst, xl_half, ex_blob, rden, zout)


def _tc2_body(xt_in, tds_in, tts_in, wres_td, btd, wres_tt, btt,
              g1, b1, g2, b2, bt_in, out_ref, acc):
    i = pl.program_id(0)

    @pl.when(i == 0)
    def _():
        acc[...] = jnp.zeros_like(acc)

    xt = xt_in[...]
    tdw = tds_in[...] + jnp.dot(xt, wres_td[...]) + btd[...]
    ttw = tts_in[...] + jnp.dot(xt, wres_tt[...]) + btt[...]
    xdu = _leaky(_ln(tdw, g1[...], b1[...]), 0.01)
    xtu = _leaky(_ln(ttw, g2[...], b2[...]), 0.01)
    blk = xt.shape[0]
    fused = jnp.concatenate([xt, xtu, xdu, jnp.ones((blk, 64), _f32)], axis=1)
    brow = bt_in[0, 0, :]
    oh = (lax.broadcasted_iota(_i32, (B, blk), 0) == brow[None, :]).astype(_f32)
    acc[...] += jnp.dot(oh, fused)
    s = acc[:, :192]
    cnt = acc[:, 192:193]
    out_ref[...] = s / jnp.maximum(cnt, 1.0)


def _tc2(xt, td_sum, tt_sum, p, b_tasks):
    blk = 1000
    grid = N // blk
    row = lambda shape: pl.BlockSpec(shape, lambda i: (i, 0))
    full = lambda shape: pl.BlockSpec(shape, lambda i: (0, 0))
    b3 = b_tasks.reshape(grid, 1, blk)
    return pl.pallas_call(
        _tc2_body,
        grid=(grid,),
        in_specs=[row((blk, 64)), row((blk, 64)), row((blk, 64)),
                  full((64, 64)), full((1, 64)), full((64, 64)), full((1, 64)),
                  full((1, 64)), full((1, 64)), full((1, 64)), full((1, 64)),
                  pl.BlockSpec((1, 1, blk), lambda i: (i, 0, 0))],
        out_specs=pl.BlockSpec((B, 192), lambda i: (0, 0)),
        out_shape=jax.ShapeDtypeStruct((B, 192), _f32),
        scratch_shapes=[pltpu.VMEM((B, 256), _f32)],
    )(xt, td_sum, tt_sum,
      p['td_Wres'], p['td_bias'].reshape(1, -1),
      p['tt_Wres'], p['tt_bias'].reshape(1, -1),
      p['ln1_g'].reshape(1, -1), p['ln1_b'].reshape(1, -1),
      p['ln2_g'].reshape(1, -1), p['ln2_b'].reshape(1, -1), b3)


def _conv_edge_phase(src, dst, xl_lo, xl_hi, xr_pad, att_perm, zden, zout):
    ex_blob, den_par = _sc_a(src, dst, xl_lo, xl_hi, xr_pad, att_perm, zden)
    den = den_par[0] + den_par[1]
    rden = (1.0 / H) / (den + 1e-16)
    out_lo = _sc_b(src, dst, xl_lo, ex_blob, rden, zout)
    out_hi = _sc_b(src, dst, xl_hi, ex_blob, rden, zout)
    lo = (out_lo[:NPW] + out_lo[NPW:]).reshape(NP, 32)
    hi = (out_hi[:NPW] + out_hi[NPW:]).reshape(NP, 32)
    return jnp.concatenate([lo[:N], hi[:N]], axis=1)


def kernel(x_tasks, x_data, params, read_edge_index, read_edge_attr, tt_edge_index, b_tasks):
    p = params
    xt, td_lo, td_hi, td_xr, tt_lo, tt_hi, tt_xr = _tc1(x_tasks, x_data, p)

    pad_src = jnp.zeros((E_PAD - E,), _i32)
    pad_dst = jnp.full((E_PAD - E,), N, _i32)
    mask = read_edge_attr[:, 0] != 0
    src_td = jnp.concatenate([read_edge_index[0], pad_src])
    dst_td = jnp.concatenate([jnp.where(mask, read_edge_index[1], N), pad_dst])
    src_tt = jnp.concatenate([tt_edge_index[0], pad_src])
    dst_tt = jnp.concatenate([tt_edge_index[1], pad_dst])

    zrow = jnp.zeros((1, 256), _f32)
    td_xr_pad = jnp.concatenate([td_xr, zrow], axis=0)
    tt_xr_pad = jnp.concatenate([tt_xr, zrow], axis=0)

    att_td = p['td_att'].reshape(-1)[_PERM].reshape(16, 16)
    att_tt = p['tt_att'].reshape(-1)[_PERM].reshape(16, 16)
    zden = jnp.zeros((ZS,), _f32)
    zout = jnp.zeros((SL2,), _f32)

    td_sum = _conv_edge_phase(src_td, dst_td, td_lo, td_hi, td_xr_pad, att_td, zden, zout)
    tt_sum = _conv_edge_phase(src_tt, dst_tt, tt_lo, tt_hi, tt_xr_pad, att_tt, zden, zout)

    return _tc2(xt, td_sum, tt_sum, p, b_tasks)


# speedup vs baseline: 14.0646x; 1.2369x over previous
import functools

import jax
import jax.numpy as jnp
import numpy as np
from jax import lax
from jax.experimental import pallas as pl
from jax.experimental.pallas import tpu as pltpu
from jax.experimental.pallas import tpu_sc as plsc

H = 4
C = 64
HID = 64
B = 8
N = 50000
F = 64
E = 800000

NW = 32
G = 128
EPW = 25088
E_PAD = NW * EPW
NCH = EPW // G
NG = G // 16
NT4 = 4 * G // 16
NP = 51200
RPS = NP // 16
DEN_F = 4 * NP
ZS = DEN_F // 16

_f32 = jnp.float32
_i32 = jnp.int32


def _leaky(x, s):
    return jnp.maximum(x, s * x)


def _ln(x, g, b, eps=1e-5):
    m = x.mean(axis=-1, keepdims=True)
    v = ((x - m) ** 2).mean(axis=-1, keepdims=True)
    return (x - m) / jnp.sqrt(v + eps) * g + b


_q = np.arange(H * C)
_PERM = ((_q % 128) // 32) * 64 + (_q // 128) * 32 + (_q % 32)


def _tc1_body(xt_in, xd_in, stW, stb, stg, stbe, sdW, sdb, sdg, sdbe,
              wl_td, wr_td, wl_tt, wr_tt,
              xt_o, tdlo_o, tdhi_o, tdxr_o, ttlo_o, tthi_o, ttxr_o):
    xt = _leaky(_ln(jnp.dot(xt_in[...], stW[...]) + stb[...], stg[...], stbe[...]), 0.01)
    xd = _leaky(_ln(jnp.dot(xd_in[...], sdW[...]) + sdb[...], sdg[...], sdbe[...]), 0.01)
    xl_td = jnp.dot(xd, wl_td[...])
    xl_tt = jnp.dot(xt, wl_tt[...])
    xt_o[...] = xt
    tdlo_o[...] = xl_td[:, :128]
    tdhi_o[...] = xl_td[:, 128:]
    tdxr_o[...] = jnp.dot(xt, wr_td[...])
    ttlo_o[...] = xl_tt[:, :128]
    tthi_o[...] = xl_tt[:, 128:]
    ttxr_o[...] = jnp.dot(xt, wr_tt[...])


def _tc1(x_tasks, x_data, p):
    blk = 1000
    grid = N // blk
    row = lambda shape: pl.BlockSpec(shape, lambda i: (i, 0))
    full = lambda shape: pl.BlockSpec(shape, lambda i: (0, 0))
    outs = pl.pallas_call(
        _tc1_body,
        grid=(grid,),
        in_specs=[row((blk, F)), row((blk, F)),
                  full((F, HID)), full((1, HID)), full((1, HID)), full((1, HID)),
                  full((F, HID)), full((1, HID)), full((1, HID)), full((1, HID)),
                  full((HID, 256)), full((HID, 256)), full((HID, 256)), full((HID, 256))],
        out_specs=[row((blk, 64)), row((blk, 128)), row((blk, 128)), row((blk, 256)),
                   row((blk, 128)), row((blk, 128)), row((blk, 256))],
        out_shape=[jax.ShapeDtypeStruct((N, 64), _f32),
                   jax.ShapeDtypeStruct((N, 128), _f32),
                   jax.ShapeDtypeStruct((N, 128), _f32),
                   jax.ShapeDtypeStruct((N, 256), _f32),
                   jax.ShapeDtypeStruct((N, 128), _f32),
                   jax.ShapeDtypeStruct((N, 128), _f32),
                   jax.ShapeDtypeStruct((N, 256), _f32)],
    )(x_tasks, x_data,
      p['stem_t_W'], p['stem_t_b'].reshape(1, -1), p['stem_t_g'].reshape(1, -1),
      p['stem_t_beta'].reshape(1, -1),
      p['stem_d_W'], p['stem_d_b'].reshape(1, -1), p['stem_d_g'].reshape(1, -1),
      p['stem_d_beta'].reshape(1, -1),
      p['td_Wl'][:, _PERM], p['td_Wr'][:, _PERM],
      p['tt_Wl'][:, _PERM], p['tt_Wr'][:, _PERM])
    return outs


def _sc_a_body(src_h, dst_h, xllo_h, xlhi_h, xr_h, att_h, zden_h,
               ex_o, den_o,
               src_v, dst_v, xi_lo, xi_hi, xj, lg, idx_all, att_v, stg, den_s):
    core = lax.axis_index("c")
    sub = lax.axis_index("s")
    wid = sub * 2 + core
    pltpu.sync_copy(att_h, att_v)
    pltpu.sync_copy(zden_h, den_s.at[pl.ds(sub * ZS, ZS)])
    plsc.subcore_barrier()
    base_w = wid * EPW

    def chunk(ci, carry):
        base = base_w + ci * G
        pltpu.sync_copy(src_h.at[pl.ds(base, G)], src_v)
        pltpu.sync_copy(dst_h.at[pl.ds(base, G)], dst_v)
        pltpu.sync_copy(xllo_h.at[src_v], xi_lo)
        pltpu.sync_copy(xlhi_h.at[src_v], xi_hi)
        pltpu.sync_copy(xr_h.at[dst_v], xj)

        def group(g, c2):
            for i in range(16):
                e = 16 * g + i
                for h in range(H):
                    acc = None
                    for jj in range(2):
                        j = 2 * h + jj
                        for half, buf in ((0, xi_lo), (1, xi_hi)):
                            a = buf[e, pl.ds(16 * j, 16)]
                            bv = xj[e, pl.ds(128 * half + 16 * j, 16)]
                            s = a + bv
                            lk = jnp.maximum(s, 0.2 * s)
                            t = att_v[half * 8 + j, :] * lk
                            acc = t if acc is None else acc + t
                    stg[pl.ds(h * 256 + 16 * i, 16)] = acc
            for h in range(H):
                tot = None
                for j in range(16):
                    idxv = lax.iota(_i32, 16) * 16 + (h * 256 + j)
                    t = plsc.load_gather(stg, [idxv])
                    tot = t if tot is None else tot + t
                lg[pl.ds(h * G + 16 * g, 16)] = tot
            return c2

        lax.fori_loop(0, G // 16, group, 0)
        for t in range(NT4):
            v = lg[pl.ds(16 * t, 16)]
            lg[pl.ds(16 * t, 16)] = jnp.exp(v)
        for t in range(NT4):
            dvec = dst_v[pl.ds((t % NG) * 16, 16)]
            idx_all[pl.ds(16 * t, 16)] = dvec + (t // NG) * NP
        pltpu.sync_copy(lg, ex_o.at[pl.ds(base * 4, 4 * G)])
        pltpu.sync_copy(lg, den_s.at[idx_all], add=True)
        return carry

    lax.fori_loop(0, NCH, chunk, 0)
    plsc.subcore_barrier()
    pltpu.sync_copy(den_s.at[pl.ds(sub * ZS, ZS)], den_o.at[core, pl.ds(sub * ZS, ZS)])


def _sc_a(src, dst, xl_lo, xl_hi, xr_pad, att_perm, zden):
    mesh = plsc.VectorSubcoreMesh(core_axis_name="c", subcore_axis_name="s")
    k = pl.kernel(
        _sc_a_body,
        out_type=(jax.ShapeDtypeStruct((4 * E_PAD,), _f32),
                  jax.ShapeDtypeStruct((2, DEN_F), _f32)),
        mesh=mesh,
        scratch_types=[
            pltpu.VMEM((G,), _i32),
            pltpu.VMEM((G,), _i32),
            pltpu.VMEM((G, 128), _f32),
            pltpu.VMEM((G, 128), _f32),
            pltpu.VMEM((G, 256), _f32),
            pltpu.VMEM((4 * G,), _f32),
            pltpu.VMEM((4 * G,), _i32),
            pltpu.VMEM((16, 16), _f32),
            pltpu.VMEM((1024,), _f32),
            pltpu.VMEM_SHARED((DEN_F,), _f32),
        ],
        compiler_params=pltpu.CompilerParams(needs_layout_passes=False),
    )
    return k(src, dst, xl_lo, xl_hi, xr_pad, att_perm, zden)


def _splat(v, i):
    idx = jnp.full((16, 1), i, _i32)
    dn = lax.GatherDimensionNumbers(offset_dims=(), collapsed_slice_dims=(0,),
                                    start_index_map=(0,))
    return lax.gather(v, idx, dn, (1,),
                      mode=lax.GatherScatterMode.PROMISE_IN_BOUNDS)


NPW = NP * 32
SL2 = NPW // 16


def _sc_b_body(src_h, dst_h, xl_h, ex_h, rden_h, zout_h,
               out_o,
               src_v, dst_v, xi, exv, rdi, rdv, av, val2, idx2, out_s):
    core = lax.axis_index("c")
    sub = lax.axis_index("s")
    wid = sub * 2 + core
    pltpu.sync_copy(zout_h, out_s.at[pl.ds(sub * SL2, SL2)])
    plsc.subcore_barrier()
    base_w = wid * EPW

    def chunk(ci, carry):
        base = base_w + ci * G
        pltpu.sync_copy(src_h.at[pl.ds(base, G)], src_v)
        pltpu.sync_copy(dst_h.at[pl.ds(base, G)], dst_v)
        pltpu.sync_copy(xl_h.at[src_v], xi)
        pltpu.sync_copy(ex_h.at[pl.ds(base * 4, 4 * G)], exv)
        for t in range(NT4):
            dvec = dst_v[pl.ds((t % NG) * 16, 16)]
            rdi[pl.ds(16 * t, 16)] = dvec + (t // NG) * NP
        pltpu.sync_copy(rden_h.at[rdi], rdv)
        for t in range(NT4):
            av[pl.ds(16 * t, 16)] = exv[pl.ds(16 * t, 16)] * rdv[pl.ds(16 * t, 16)]
        iot = lax.iota(_i32, 16)

        def group(g, c2):
            vh = [av[pl.ds(h * G + 16 * g, 16)] for h in range(H)]
            dvec = dst_v[pl.ds(16 * g, 16)]
            for i in range(16):
                e = 16 * g + i
                sp = [_splat(vh[h], i) for h in range(H)]
                d32 = _splat(dvec, i) * 32
                for jj in range(2):
                    acc = None
                    for h in range(H):
                        t = sp[h] * xi[e, pl.ds(h * 32 + 16 * jj, 16)]
                        acc = t if acc is None else acc + t
                    val2[pl.ds(e * 32 + 16 * jj, 16)] = acc
                    idx2[pl.ds(e * 32 + 16 * jj, 16)] = d32 + (iot + 16 * jj)
            return c2

        lax.fori_loop(0, G // 16, group, 0)
        pltpu.sync_copy(val2, out_s.at[idx2], add=True)
        return carry

    lax.fori_loop(0, NCH, chunk, 0)
    plsc.subcore_barrier()
    pltpu.sync_copy(out_s.at[pl.ds(sub * SL2, SL2)],
                    out_o.at[pl.ds(core * NPW + sub * SL2, SL2)])


def _sc_b(src, dst, xl_half, ex_blob, rden, zout):
    mesh = plsc.VectorSubcoreMesh(core_axis_name="c", subcore_axis_name="s")
    k = pl.kernel(
        _sc_b_body,
        out_type=jax.ShapeDtypeStruct((2 * NPW,), _f32),
        mesh=mesh,
        scratch_types=[
            pltpu.VMEM((G,), _i32),
            pltpu.VMEM((G,), _i32),
            pltpu.VMEM((G, 128), _f32),
            pltpu.VMEM((4 * G,), _f32),
            pltpu.VMEM((4 * G,), _i32),
            pltpu.VMEM((4 * G,), _f32),
            pltpu.VMEM((4 * G,), _f32),
            pltpu.VMEM((32 * G,), _f32),
            pltpu.VMEM((32 * G,), _i32),
            pltpu.VMEM_SHARED((NPW,), _f32),
        ],
        compiler_params=pltpu.CompilerParams(needs_layout_passes=False),
    )
    return k(src, dst, xl_half, ex_blob, rden, zout)


def _tc2_body(xt_in, tds_in, tts_in, wres_td, btd, wres_tt, btt,
              g1, b1, g2, b2, bt_in, out_ref, acc):
    i = pl.program_id(0)

    @pl.when(i == 0)
    def _():
        acc[...] = jnp.zeros_like(acc)

    xt = xt_in[...]
    tdw = tds_in[...] + jnp.dot(xt, wres_td[...]) + btd[...]
    ttw = tts_in[...] + jnp.dot(xt, wres_tt[...]) + btt[...]
    xdu = _leaky(_ln(tdw, g1[...], b1[...]), 0.01)
    xtu = _leaky(_ln(ttw, g2[...], b2[...]), 0.01)
    blk = xt.shape[0]
    fused = jnp.concatenate([xt, xtu, xdu, jnp.ones((blk, 64), _f32)], axis=1)
    brow = bt_in[0, 0, :]
    oh = (lax.broadcasted_iota(_i32, (B, blk), 0) == brow[None, :]).astype(_f32)
    acc[...] += jnp.dot(oh, fused)
    s = acc[:, :192]
    cnt = acc[:, 192:193]
    out_ref[...] = s / jnp.maximum(cnt, 1.0)


def _tc2(xt, td_sum, tt_sum, p, b_tasks):
    blk = 1000
    grid = N // blk
    row = lambda shape: pl.BlockSpec(shape, lambda i: (i, 0))
    full = lambda shape: pl.BlockSpec(shape, lambda i: (0, 0))
    b3 = b_tasks.reshape(grid, 1, blk)
    return pl.pallas_call(
        _tc2_body,
        grid=(grid,),
        in_specs=[row((blk, 64)), row((blk, 64)), row((blk, 64)),
                  full((64, 64)), full((1, 64)), full((64, 64)), full((1, 64)),
                  full((1, 64)), full((1, 64)), full((1, 64)), full((1, 64)),
                  pl.BlockSpec((1, 1, blk), lambda i: (i, 0, 0))],
        out_specs=pl.BlockSpec((B, 192), lambda i: (0, 0)),
        out_shape=jax.ShapeDtypeStruct((B, 192), _f32),
        scratch_shapes=[pltpu.VMEM((B, 256), _f32)],
    )(xt, td_sum, tt_sum,
      p['td_Wres'], p['td_bias'].reshape(1, -1),
      p['tt_Wres'], p['tt_bias'].reshape(1, -1),
      p['ln1_g'].reshape(1, -1), p['ln1_b'].reshape(1, -1),
      p['ln2_g'].reshape(1, -1), p['ln2_b'].reshape(1, -1), b3)


def _conv_edge_phase(src, dst, xl_lo, xl_hi, xr_pad, att_perm, zden, zout):
    ex_blob, den_par = _sc_a(src, dst, xl_lo, xl_hi, xr_pad, att_perm, zden)
    den = den_par[0] + den_par[1]
    rden = (1.0 / H) / (den + 1e-16)
    out_lo = _sc_b(src, dst, xl_lo, ex_blob, rden, zout)
    out_hi = _sc_b(src, dst, xl_hi, ex_blob, rden, zout)
    lo = (out_lo[:NPW] + out_lo[NPW:]).reshape(NP, 32)
    hi = (out_hi[:NPW] + out_hi[NPW:]).reshape(NP, 32)
    return jnp.concatenate([lo[:N], hi[:N]], axis=1)


def kernel(x_tasks, x_data, params, read_edge_index, read_edge_attr, tt_edge_index, b_tasks):
    p = params
    xt, td_lo, td_hi, td_xr, tt_lo, tt_hi, tt_xr = _tc1(x_tasks, x_data, p)

    pad_src = jnp.zeros((E_PAD - E,), _i32)
    pad_dst = jnp.full((E_PAD - E,), N, _i32)
    mask = read_edge_attr[:, 0] != 0
    src_td = jnp.concatenate([read_edge_index[0], pad_src])
    dst_td = jnp.concatenate([jnp.where(mask, read_edge_index[1], N), pad_dst])
    src_tt = jnp.concatenate([tt_edge_index[0], pad_src])
    dst_tt = jnp.concatenate([tt_edge_index[1], pad_dst])

    zrow = jnp.zeros((1, 256), _f32)
    td_xr_pad = jnp.concatenate([td_xr, zrow], axis=0)
    tt_xr_pad = jnp.concatenate([tt_xr, zrow], axis=0)

    att_td = p['td_att'].reshape(-1)[_PERM].reshape(16, 16)
    att_tt = p['tt_att'].reshape(-1)[_PERM].reshape(16, 16)
    zden = jnp.zeros((ZS,), _f32)
    zout = jnp.zeros((SL2,), _f32)

    td_sum = _conv_edge_phase(src_td, dst_td, td_lo, td_hi, td_xr_pad, att_td, zden, zout)
    tt_sum = _conv_edge_phase(src_tt, dst_tt, tt_lo, tt_hi, tt_xr_pad, att_tt, zden, zout)

    return _tc2(xt, td_sum, tt_sum, p, b_tasks)


# speedup vs baseline: 16.2637x; 1.1564x over previous
import functools

import jax
import jax.numpy as jnp
import numpy as np
from jax import lax
from jax.experimental import pallas as pl
from jax.experimental.pallas import tpu as pltpu
from jax.experimental.pallas import tpu_sc as plsc

H = 4
C = 64
HID = 64
B = 8
N = 50000
F = 64
E = 800000

NW = 32
G = 128
EPW = 25088
E_PAD = NW * EPW
NCH = EPW // G
NG = G // 16
NT4 = 4 * G // 16
NP = 51200
RPS = NP // 16
DEN_F = 4 * NP
ZS = DEN_F // 16

_f32 = jnp.float32
_i32 = jnp.int32


def _leaky(x, s):
    return jnp.maximum(x, s * x)


def _ln(x, g, b, eps=1e-5):
    m = x.mean(axis=-1, keepdims=True)
    v = ((x - m) ** 2).mean(axis=-1, keepdims=True)
    return (x - m) / jnp.sqrt(v + eps) * g + b


_q = np.arange(H * C)
_PERM = ((_q % 128) // 32) * 64 + (_q // 128) * 32 + (_q % 32)


def _tc1_body(xt_in, xd_in, stW, stb, stg, stbe, sdW, sdb, sdg, sdbe,
              wl_td, wr_td, wl_tt, wr_tt,
              xt_o, tdlo_o, tdhi_o, tdxr_o, ttlo_o, tthi_o, ttxr_o):
    xt = _leaky(_ln(jnp.dot(xt_in[...], stW[...]) + stb[...], stg[...], stbe[...]), 0.01)
    xd = _leaky(_ln(jnp.dot(xd_in[...], sdW[...]) + sdb[...], sdg[...], sdbe[...]), 0.01)
    xl_td = jnp.dot(xd, wl_td[...])
    xl_tt = jnp.dot(xt, wl_tt[...])
    xt_o[...] = xt
    tdlo_o[...] = xl_td[:, :128]
    tdhi_o[...] = xl_td[:, 128:]
    tdxr_o[...] = jnp.dot(xt, wr_td[...])
    ttlo_o[...] = xl_tt[:, :128]
    tthi_o[...] = xl_tt[:, 128:]
    ttxr_o[...] = jnp.dot(xt, wr_tt[...])


def _tc1(x_tasks, x_data, p):
    blk = 1000
    grid = N // blk
    row = lambda shape: pl.BlockSpec(shape, lambda i: (i, 0))
    full = lambda shape: pl.BlockSpec(shape, lambda i: (0, 0))
    outs = pl.pallas_call(
        _tc1_body,
        grid=(grid,),
        in_specs=[row((blk, F)), row((blk, F)),
                  full((F, HID)), full((1, HID)), full((1, HID)), full((1, HID)),
                  full((F, HID)), full((1, HID)), full((1, HID)), full((1, HID)),
                  full((HID, 256)), full((HID, 256)), full((HID, 256)), full((HID, 256))],
        out_specs=[row((blk, 64)), row((blk, 128)), row((blk, 128)), row((blk, 256)),
                   row((blk, 128)), row((blk, 128)), row((blk, 256))],
        out_shape=[jax.ShapeDtypeStruct((N, 64), _f32),
                   jax.ShapeDtypeStruct((N, 128), _f32),
                   jax.ShapeDtypeStruct((N, 128), _f32),
                   jax.ShapeDtypeStruct((N, 256), _f32),
                   jax.ShapeDtypeStruct((N, 128), _f32),
                   jax.ShapeDtypeStruct((N, 128), _f32),
                   jax.ShapeDtypeStruct((N, 256), _f32)],
    )(x_tasks, x_data,
      p['stem_t_W'], p['stem_t_b'].reshape(1, -1), p['stem_t_g'].reshape(1, -1),
      p['stem_t_beta'].reshape(1, -1),
      p['stem_d_W'], p['stem_d_b'].reshape(1, -1), p['stem_d_g'].reshape(1, -1),
      p['stem_d_beta'].reshape(1, -1),
      p['td_Wl'][:, _PERM], p['td_Wr'][:, _PERM],
      p['tt_Wl'][:, _PERM], p['tt_Wr'][:, _PERM])
    return outs


def _sc_a_body(src_h, dst_h, xllo_h, xlhi_h, xr_h, att_h, zden_h,
               ex_o, den_o,
               src_v, dst_v, xi_lo, xi_hi, xj, lg, idx_all, att_v, stg, den_s,
               sem1, sem2, sem3):
    core = lax.axis_index("c")
    sub = lax.axis_index("s")
    wid = sub * 2 + core
    pltpu.sync_copy(att_h, att_v)
    pltpu.sync_copy(zden_h, den_s.at[pl.ds(sub * ZS, ZS)])
    plsc.subcore_barrier()
    base_w = wid * EPW

    def chunk(ci, carry):
        base = base_w + ci * G
        c1 = pltpu.async_copy(src_h.at[pl.ds(base, G)], src_v, sem1)
        c2 = pltpu.async_copy(dst_h.at[pl.ds(base, G)], dst_v, sem2)
        c1.wait()
        c2.wait()
        g1 = pltpu.async_copy(xllo_h.at[src_v], xi_lo, sem1)
        g2 = pltpu.async_copy(xlhi_h.at[src_v], xi_hi, sem2)
        g3 = pltpu.async_copy(xr_h.at[dst_v], xj, sem3)
        g1.wait()
        g2.wait()
        g3.wait()

        def group(g, c2):
            for i in range(16):
                e = 16 * g + i
                for h in range(H):
                    acc = None
                    for jj in range(2):
                        j = 2 * h + jj
                        for half, buf in ((0, xi_lo), (1, xi_hi)):
                            a = buf[e, pl.ds(16 * j, 16)]
                            bv = xj[e, pl.ds(128 * half + 16 * j, 16)]
                            s = a + bv
                            lk = jnp.maximum(s, 0.2 * s)
                            t = att_v[half * 8 + j, :] * lk
                            acc = t if acc is None else acc + t
                    stg[pl.ds(h * 256 + 16 * i, 16)] = acc
            for h in range(H):
                tot = None
                for j in range(16):
                    idxv = lax.iota(_i32, 16) * 16 + (h * 256 + j)
                    t = plsc.load_gather(stg, [idxv])
                    tot = t if tot is None else tot + t
                lg[pl.ds(h * G + 16 * g, 16)] = tot
            return c2

        lax.fori_loop(0, G // 16, group, 0)
        for t in range(NT4):
            v = lg[pl.ds(16 * t, 16)]
            lg[pl.ds(16 * t, 16)] = jnp.exp(v)
        for t in range(NT4):
            dvec = dst_v[pl.ds((t % NG) * 16, 16)]
            idx_all[pl.ds(16 * t, 16)] = dvec + (t // NG) * NP
        w1 = pltpu.async_copy(lg, ex_o.at[pl.ds(base * 4, 4 * G)], sem1)
        w2 = pltpu.async_copy(lg, den_s.at[idx_all], sem2, add=True)
        w1.wait()
        w2.wait()
        return carry

    lax.fori_loop(0, NCH, chunk, 0)
    plsc.subcore_barrier()
    pltpu.sync_copy(den_s.at[pl.ds(sub * ZS, ZS)], den_o.at[core, pl.ds(sub * ZS, ZS)])


def _sc_a(src, dst, xl_lo, xl_hi, xr_pad, att_perm, zden):
    mesh = plsc.VectorSubcoreMesh(core_axis_name="c", subcore_axis_name="s")
    k = pl.kernel(
        _sc_a_body,
        out_type=(jax.ShapeDtypeStruct((4 * E_PAD,), _f32),
                  jax.ShapeDtypeStruct((2, DEN_F), _f32)),
        mesh=mesh,
        scratch_types=[
            pltpu.VMEM((G,), _i32),
            pltpu.VMEM((G,), _i32),
            pltpu.VMEM((G, 128), _f32),
            pltpu.VMEM((G, 128), _f32),
            pltpu.VMEM((G, 256), _f32),
            pltpu.VMEM((4 * G,), _f32),
            pltpu.VMEM((4 * G,), _i32),
            pltpu.VMEM((16, 16), _f32),
            pltpu.VMEM((1024,), _f32),
            pltpu.VMEM_SHARED((DEN_F,), _f32),
            pltpu.SemaphoreType.DMA,
            pltpu.SemaphoreType.DMA,
            pltpu.SemaphoreType.DMA,
        ],
        compiler_params=pltpu.CompilerParams(needs_layout_passes=False),
    )
    return k(src, dst, xl_lo, xl_hi, xr_pad, att_perm, zden)


def _splat(v, i):
    idx = jnp.full((16, 1), i, _i32)
    dn = lax.GatherDimensionNumbers(offset_dims=(), collapsed_slice_dims=(0,),
                                    start_index_map=(0,))
    return lax.gather(v, idx, dn, (1,),
                      mode=lax.GatherScatterMode.PROMISE_IN_BOUNDS)


NPW = NP * 32
SL2 = NPW // 16


def _sc_b_body(src_h, dst_h, xl_h, ex_h, rden_h, zout_h,
               out_o,
               src_v, dst_v, xi, exv, rdi, rdv, av, val2, idx2, out_s,
               sem1, sem2, sem3):
    core = lax.axis_index("c")
    sub = lax.axis_index("s")
    wid = sub * 2 + core
    pltpu.sync_copy(zout_h, out_s.at[pl.ds(sub * SL2, SL2)])
    plsc.subcore_barrier()
    base_w = wid * EPW

    def chunk(ci, carry):
        base = base_w + ci * G
        c1 = pltpu.async_copy(src_h.at[pl.ds(base, G)], src_v, sem1)
        c2 = pltpu.async_copy(dst_h.at[pl.ds(base, G)], dst_v, sem2)
        c1.wait()
        c2.wait()
        g1 = pltpu.async_copy(xl_h.at[src_v], xi, sem1)
        g2 = pltpu.async_copy(ex_h.at[pl.ds(base * 4, 4 * G)], exv, sem2)
        for t in range(NT4):
            dvec = dst_v[pl.ds((t % NG) * 16, 16)]
            rdi[pl.ds(16 * t, 16)] = dvec + (t // NG) * NP
        g3 = pltpu.async_copy(rden_h.at[rdi], rdv, sem3)
        g1.wait()
        g2.wait()
        g3.wait()
        for t in range(NT4):
            av[pl.ds(16 * t, 16)] = exv[pl.ds(16 * t, 16)] * rdv[pl.ds(16 * t, 16)]
        iot = lax.iota(_i32, 16)

        def group(g, c2):
            vh = [av[pl.ds(h * G + 16 * g, 16)] for h in range(H)]
            dvec = dst_v[pl.ds(16 * g, 16)]
            for i in range(16):
                e = 16 * g + i
                sp = [_splat(vh[h], i) for h in range(H)]
                d32 = _splat(dvec, i) * 32
                for jj in range(2):
                    acc = None
                    for h in range(H):
                        t = sp[h] * xi[e, pl.ds(h * 32 + 16 * jj, 16)]
                        acc = t if acc is None else acc + t
                    val2[pl.ds(e * 32 + 16 * jj, 16)] = acc
                    idx2[pl.ds(e * 32 + 16 * jj, 16)] = d32 + (iot + 16 * jj)
            return c2

        lax.fori_loop(0, G // 16, group, 0)
        pltpu.sync_copy(val2, out_s.at[idx2], add=True)
        return carry

    lax.fori_loop(0, NCH, chunk, 0)
    plsc.subcore_barrier()
    pltpu.sync_copy(out_s.at[pl.ds(sub * SL2, SL2)],
                    out_o.at[pl.ds(core * NPW + sub * SL2, SL2)])


def _sc_b(src, dst, xl_half, ex_blob, rden, zout):
    mesh = plsc.VectorSubcoreMesh(core_axis_name="c", subcore_axis_name="s")
    k = pl.kernel(
        _sc_b_body,
        out_type=jax.ShapeDtypeStruct((2 * NPW,), _f32),
        mesh=mesh,
        scratch_types=[
            pltpu.VMEM((G,), _i32),
            pltpu.VMEM((G,), _i32),
            pltpu.VMEM((G, 128), _f32),
            pltpu.VMEM((4 * G,), _f32),
            pltpu.VMEM((4 * G,), _i32),
            pltpu.VMEM((4 * G,), _f32),
            pltpu.VMEM((4 * G,), _f32),
            pltpu.VMEM((32 * G,), _f32),
            pltpu.VMEM((32 * G,), _i32),
            pltpu.VMEM_SHARED((NPW,), _f32),
            pltpu.SemaphoreType.DMA,
            pltpu.SemaphoreType.DMA,
            pltpu.SemaphoreType.DMA,
        ],
        compiler_params=pltpu.CompilerParams(needs_layout_passes=False),
    )
    return k(src, dst, xl_half, ex_blob, rden, zout)


def _tc2_body(xt_in, tds_in, tts_in, wres_td, btd, wres_tt, btt,
              g1, b1, g2, b2, bt_in, out_ref, acc):
    i = pl.program_id(0)

    @pl.when(i == 0)
    def _():
        acc[...] = jnp.zeros_like(acc)

    xt = xt_in[...]
    tdw = tds_in[...] + jnp.dot(xt, wres_td[...]) + btd[...]
    ttw = tts_in[...] + jnp.dot(xt, wres_tt[...]) + btt[...]
    xdu = _leaky(_ln(tdw, g1[...], b1[...]), 0.01)
    xtu = _leaky(_ln(ttw, g2[...], b2[...]), 0.01)
    blk = xt.shape[0]
    fused = jnp.concatenate([xt, xtu, xdu, jnp.ones((blk, 64), _f32)], axis=1)
    brow = bt_in[0, 0, :]
    oh = (lax.broadcasted_iota(_i32, (B, blk), 0) == brow[None, :]).astype(_f32)
    acc[...] += jnp.dot(oh, fused)
    s = acc[:, :192]
    cnt = acc[:, 192:193]
    out_ref[...] = s / jnp.maximum(cnt, 1.0)


def _tc2(xt, td_sum, tt_sum, p, b_tasks):
    blk = 1000
    grid = N // blk
    row = lambda shape: pl.BlockSpec(shape, lambda i: (i, 0))
    full = lambda shape: pl.BlockSpec(shape, lambda i: (0, 0))
    b3 = b_tasks.reshape(grid, 1, blk)
    return pl.pallas_call(
        _tc2_body,
        grid=(grid,),
        in_specs=[row((blk, 64)), row((blk, 64)), row((blk, 64)),
                  full((64, 64)), full((1, 64)), full((64, 64)), full((1, 64)),
                  full((1, 64)), full((1, 64)), full((1, 64)), full((1, 64)),
                  pl.BlockSpec((1, 1, blk), lambda i: (i, 0, 0))],
        out_specs=pl.BlockSpec((B, 192), lambda i: (0, 0)),
        out_shape=jax.ShapeDtypeStruct((B, 192), _f32),
        scratch_shapes=[pltpu.VMEM((B, 256), _f32)],
    )(xt, td_sum, tt_sum,
      p['td_Wres'], p['td_bias'].reshape(1, -1),
      p['tt_Wres'], p['tt_bias'].reshape(1, -1),
      p['ln1_g'].reshape(1, -1), p['ln1_b'].reshape(1, -1),
      p['ln2_g'].reshape(1, -1), p['ln2_b'].reshape(1, -1), b3)


def _conv_edge_phase(src, dst, xl_lo, xl_hi, xr_pad, att_perm, zden, zout):
    ex_blob, den_par = _sc_a(src, dst, xl_lo, xl_hi, xr_pad, att_perm, zden)
    den = den_par[0] + den_par[1]
    rden = (1.0 / H) / (den + 1e-16)
    out_lo = _sc_b(src, dst, xl_lo, ex_blob, rden, zout)
    out_hi = _sc_b(src, dst, xl_hi, ex_blob, rden, zout)
    lo = (out_lo[:NPW] + out_lo[NPW:]).reshape(NP, 32)
    hi = (out_hi[:NPW] + out_hi[NPW:]).reshape(NP, 32)
    return jnp.concatenate([lo[:N], hi[:N]], axis=1)


def kernel(x_tasks, x_data, params, read_edge_index, read_edge_attr, tt_edge_index, b_tasks):
    p = params
    xt, td_lo, td_hi, td_xr, tt_lo, tt_hi, tt_xr = _tc1(x_tasks, x_data, p)

    pad_src = jnp.zeros((E_PAD - E,), _i32)
    pad_dst = jnp.full((E_PAD - E,), N, _i32)
    mask = read_edge_attr[:, 0] != 0
    src_td = jnp.concatenate([read_edge_index[0], pad_src])
    dst_td = jnp.concatenate([jnp.where(mask, read_edge_index[1], N), pad_dst])
    src_tt = jnp.concatenate([tt_edge_index[0], pad_src])
    dst_tt = jnp.concatenate([tt_edge_index[1], pad_dst])

    zrow = jnp.zeros((1, 256), _f32)
    td_xr_pad = jnp.concatenate([td_xr, zrow], axis=0)
    tt_xr_pad = jnp.concatenate([tt_xr, zrow], axis=0)

    att_td = p['td_att'].reshape(-1)[_PERM].reshape(16, 16)
    att_tt = p['tt_att'].reshape(-1)[_PERM].reshape(16, 16)
    zden = jnp.zeros((ZS,), _f32)
    zout = jnp.zeros((SL2,), _f32)

    td_sum = _conv_edge_phase(src_td, dst_td, td_lo, td_hi, td_xr_pad, att_td, zden, zout)
    tt_sum = _conv_edge_phase(src_tt, dst_tt, tt_lo, tt_hi, tt_xr_pad, att_tt, zden, zout)

    return _tc2(xt, td_sum, tt_sum, p, b_tasks)
